# Initial kernel scaffold; baseline (speedup 1.0000x reference)
import jax
import jax.numpy as jnp
from jax.experimental import pallas as pl


def kernel(pred, target, target_identifiers):
    raise NotImplementedError("write your pallas kernel here")


# speedup vs baseline: 66.1998x; 66.1998x over previous
import functools

import jax
import jax.numpy as jnp
from jax import lax
from jax.experimental import pallas as pl
from jax.experimental.pallas import tpu as pltpu
from jax.experimental.pallas import tpu_sc as plsc

NSEG = 512
NPOS = 8192
NNODES = 10000
DIM = 128
L = 16
NC = 2
NS = 16
NW = NC * NS
RPW = NPOS // NS
NG_T = RPW // L
PRED_CHUNK = 320
PRED_LAST = NNODES - PRED_CHUNK * (NW - 1)
DSTEPS = DIM // 8


def _rsqrt(s):
    i = plsc.bitcast(s, jnp.int32)
    y = plsc.bitcast(jnp.int32(0x5F3759DF) - (i >> 1), jnp.float32)
    for _ in range(3):
        y = y * (1.5 - 0.5 * s * y * y)
    return y


def _sc_body(pred_hbm, tgt_hbm, ids_hbm, out_hbm,
             ids_v, stt_v, end_v, cnt_all, recip_v, tgt_v, scale_v,
             vv, vtmp_v, out_v, sh_cnt, sh_v):
    cid = lax.axis_index("c")
    sid = lax.axis_index("s")
    wid = cid * NS + sid
    lanes = lax.iota(jnp.int32, L)
    zf = jnp.zeros((L,), jnp.float32)

    pltpu.sync_copy(ids_hbm, ids_v)

    zi = jnp.zeros((L,), jnp.int32)

    def _zero(i, _):
        stt_v[pl.ds(i * L, L)] = zi
        end_v[pl.ds(i * L, L)] = zi
        return 0

    lax.fori_loop(0, NSEG // L, _zero, 0)

    base = sid * RPW

    def _bounds(g, _):
        o = base + g * L
        gidx = o + lanes
        cur = ids_v[pl.ds(o, L)]
        prv = plsc.load_gather(ids_v, [jnp.maximum(gidx - 1, 0)])
        nxt = plsc.load_gather(ids_v, [jnp.minimum(gidx + 1, NPOS - 1)])
        m_s = (cur != prv) | (gidx == 0)
        m_e = (cur != nxt) | (gidx == NPOS - 1)
        plsc.store_scatter(stt_v, [cur], gidx, mask=m_s)
        plsc.store_scatter(end_v, [cur], gidx + 1, mask=m_e)
        return 0

    lax.fori_loop(0, NG_T, _bounds, 0)

    def _diff(i, _):
        c = (end_v[pl.ds(i * L, L)] - stt_v[pl.ds(i * L, L)]).astype(jnp.float32)
        recip_v[pl.ds(i * L, L)] = c
        return 0

    lax.fori_loop(0, NSEG // L, _diff, 0)
    pltpu.sync_copy(recip_v, sh_cnt.at[sid])
    plsc.subcore_barrier()
    pltpu.sync_copy(sh_cnt, cnt_all)

    def _recip(i, _):
        acc = zf
        for r in range(NS):
            acc = acc + cnt_all[r, pl.ds(i * L, L)]
        recip_v[pl.ds(i * L, L)] = 1.0 / (float(NSEG) * jnp.maximum(acc, 1.0))
        return 0

    lax.fori_loop(0, NSEG // L, _recip, 0)

    pltpu.sync_copy(tgt_hbm.at[pl.ds(base, RPW)], tgt_v)

    def _scales(g, _):
        o = g * L
        rows = o + lanes

        def _dstep(d8, acc):
            for k in range(8):
                dd = d8 * 8 + k
                col = jnp.full((L,), dd, jnp.int32)
                gth = plsc.load_gather(tgt_v, [rows, col])
                acc = acc + gth * gth
            return acc

        ssq = lax.fori_loop(0, DSTEPS, _dstep, zf)
        y = _rsqrt(jnp.maximum(ssq, 1e-30))
        ids16 = ids_v[pl.ds(base + o, L)]
        w16 = plsc.load_gather(recip_v, [ids16])
        scale_v[pl.ds(o, L)] = w16 * y
        return 0

    lax.fori_loop(0, NG_T, _scales, 0)

    def _rowacc(g, accs):
        o = g * L
        sc16 = scale_v[pl.ds(o, L)]
        accs = list(accs)
        for l in range(L):
            sc = jnp.full((L,), sc16[l], jnp.float32)
            for k in range(8):
                t = tgt_v[o + l, pl.ds(k * L, L)]
                accs[k] = accs[k] + sc * t
        return tuple(accs)

    accs = lax.fori_loop(0, NG_T, _rowacc, tuple(zf for _ in range(8)))
    for k in range(8):
        vv[pl.ds(k * L, L)] = accs[k]
    pltpu.sync_copy(vv, sh_v.at[sid])
    plsc.subcore_barrier()
    pltpu.sync_copy(sh_v, vtmp_v)
    for k in range(8):
        acc = zf
        for r in range(NS):
            acc = acc + vtmp_v[r, pl.ds(k * L, L)]
        vv[pl.ds(k * L, L)] = acc

    pbase = wid * PRED_CHUNK

    @pl.when(wid < NW - 1)
    def _():
        pltpu.sync_copy(pred_hbm.at[pl.ds(pbase, PRED_CHUNK)],
                        tgt_v.at[pl.ds(0, PRED_CHUNK)])

    @pl.when(wid == NW - 1)
    def _():
        pltpu.sync_copy(pred_hbm.at[pl.ds(NNODES - PRED_LAST, PRED_LAST)],
                        tgt_v.at[pl.ds(0, PRED_LAST)])

    ng = jnp.where(wid == NW - 1, PRED_LAST // L, PRED_CHUNK // L)

    def _outg(g, _):
        o = g * L
        rows = o + lanes

        def _dstep(d16, carry):
            accd, accs2 = carry
            vvec = vv[pl.ds(d16 * L, L)]
            for k in range(L):
                col = jnp.full((L,), d16 * L + k, jnp.int32)
                gth = plsc.load_gather(tgt_v, [rows, col])
                vd = jnp.full((L,), vvec[k], jnp.float32)
                accd = accd + gth * vd
                accs2 = accs2 + gth * gth
            return (accd, accs2)

        accd, accs2 = lax.fori_loop(0, DIM // L, _dstep, (zf, zf))
        y = _rsqrt(jnp.maximum(accs2, 1e-30))
        out_v[pl.ds(o, L)] = -(accd * y)
        return 0

    lax.fori_loop(0, ng, _outg, 0)

    @pl.when(wid < NW - 1)
    def _():
        pltpu.sync_copy(out_v, out_hbm.at[pl.ds(pbase, PRED_CHUNK)])

    @pl.when(wid == NW - 1)
    def _():
        pltpu.sync_copy(out_v.at[pl.ds(0, PRED_LAST)],
                        out_hbm.at[pl.ds(NNODES - PRED_LAST, PRED_LAST)])


def _build(interpret=False):
    return pl.kernel(
        _sc_body,
        out_type=jax.ShapeDtypeStruct((NNODES,), jnp.float32),
        mesh=plsc.VectorSubcoreMesh(core_axis_name="c", subcore_axis_name="s",
                                    num_cores=NC, num_subcores=NS),
        scratch_types=[
            pltpu.VMEM((NPOS,), jnp.int32),
            pltpu.VMEM((NSEG,), jnp.int32),
            pltpu.VMEM((NSEG,), jnp.int32),
            pltpu.VMEM((NS, NSEG), jnp.float32),
            pltpu.VMEM((NSEG,), jnp.float32),
            pltpu.VMEM((RPW, DIM), jnp.float32),
            pltpu.VMEM((RPW,), jnp.float32),
            pltpu.VMEM((DIM,), jnp.float32),
            pltpu.VMEM((NS, DIM), jnp.float32),
            pltpu.VMEM((PRED_CHUNK,), jnp.float32),
            pltpu.VMEM_SHARED((NS, NSEG), jnp.float32),
            pltpu.VMEM_SHARED((NS, DIM), jnp.float32),
        ],
        compiler_params=pltpu.CompilerParams(needs_layout_passes=False),
        interpret=interpret,
    )


_sc_kernel = _build()


@jax.jit
def kernel(pred, target, target_identifiers):
    ids = target_identifiers.astype(jnp.int32)
    return _sc_kernel(pred, target, ids)
